# baseline (speedup 1.0000x reference)
import functools

import jax
import jax.numpy as jnp
from jax import lax
from jax.experimental import pallas as pl
from jax.experimental.pallas import tpu as pltpu
from jax.experimental.pallas import tpu_sc as plsc

E = 64
K = 2
AUX_COEF = 0.01
T_BLK = 1024

_NC = 2
_NS = 16
_NW = _NC * _NS
_TPT = 1024
_G = 4


def _router_body(x_ref, w_ref, b_ref, p_ref, pt_ref, psum_ref, cnt_ref,
                 aux_ref):
    i = pl.program_id(0)
    n_tok_total = pl.num_programs(0) * x_ref.shape[0]
    acc = jax.lax.dot_general(
        x_ref[...], w_ref[...], (((1,), (1,)), ((), ())),
        preferred_element_type=jnp.float32)
    lb = (acc + b_ref[...].astype(jnp.float32)).astype(jnp.bfloat16)
    l = lb.astype(jnp.float32)
    e = jnp.exp(l)
    s = jnp.sum(e, axis=-1, keepdims=True)
    inv = 1.0 / s
    p = e * inv
    p_ref[...] = p
    pt_ref[...] = p.T.reshape(1, E, T_BLK)

    iota = jax.lax.broadcasted_iota(jnp.int32, p.shape, 1)
    pbits = jax.lax.bitcast_convert_type(p, jnp.int32)
    key = (pbits & ~63) | (63 - iota)
    k1 = jnp.max(key, axis=-1, keepdims=True)
    eq1 = key == k1

    ps = jnp.sum(p, axis=0, keepdims=True)
    cs = jnp.sum(eq1.astype(jnp.float32), axis=0, keepdims=True)

    @pl.when(i == 0)
    def _():
        psum_ref[...] = ps
        cnt_ref[...] = cs

    @pl.when(i > 0)
    def _():
        psum_ref[...] += ps
        cnt_ref[...] += cs

    @pl.when(i == pl.num_programs(0) - 1)
    def _():
        scale = E * AUX_COEF / (float(n_tok_total) * float(n_tok_total))
        aux_ref[...] = jnp.sum(psum_ref[...] * cnt_ref[...],
                               axis=(0, 1), keepdims=True) * scale


def _sc_route_body(pt_hbm, w1_hbm, w2_hbm, i1_hbm, i2_hbm, pt_v,
                   w1_v, w2_v, i1_v, i2_v):
    wid = lax.axis_index("s") * _NC + lax.axis_index("c")
    base = wid * _TPT
    pltpu.sync_copy(pt_hbm.at[pl.ds(wid * (_TPT * E), _TPT * E)], pt_v)
    lanes = lax.iota(jnp.int32, 16)

    def tstep(t, carry_unused):
        tok0 = t * (16 * _G)
        init = []
        for g in range(_G):
            init += [jnp.full((16,), -1.0, jnp.float32),
                     jnp.zeros((16,), jnp.int32),
                     jnp.full((16,), -2.0, jnp.float32),
                     jnp.zeros((16,), jnp.int32)]

        def inner(j, carry):
            cols = jnp.full((16,), j, jnp.int32)
            off = j * _TPT + tok0
            out = []
            for g in range(_G):
                m1, i1, m2, i2 = carry[4 * g:4 * g + 4]
                pj = pt_v[pl.ds(off + 16 * g, 16)]
                gt1 = pj > m1
                gt2 = pj > m2
                m2n = jnp.where(gt1, m1, jnp.where(gt2, pj, m2))
                i2n = jnp.where(gt1, i1, jnp.where(gt2, cols, i2))
                m1n = jnp.where(gt1, pj, m1)
                i1n = jnp.where(gt1, cols, i1)
                out += [m1n, i1n, m2n, i2n]
            return tuple(out)

        res = lax.fori_loop(0, E, inner, tuple(init))
        for g in range(_G):
            m1, i1, m2, i2 = res[4 * g:4 * g + 4]
            e2 = jnp.exp(m2 - m1)
            den = 1.0 + e2
            sl = pl.ds(tok0 + 16 * g, 16)
            w1_v[sl] = 1.0 / den
            w2_v[sl] = e2 / den
            i1_v[sl] = i1
            i2_v[sl] = i2
        return 0

    lax.fori_loop(0, _TPT // (16 * _G), tstep, 0)
    pltpu.sync_copy(w1_v, w1_hbm.at[pl.ds(base, _TPT)])
    pltpu.sync_copy(w2_v, w2_hbm.at[pl.ds(base, _TPT)])
    pltpu.sync_copy(i1_v, i1_hbm.at[pl.ds(base, _TPT)])
    pltpu.sync_copy(i2_v, i2_hbm.at[pl.ds(base, _TPT)])


def kernel(hidden_states, W, b):
    Bx, Sx, H = hidden_states.shape
    n = Bx * Sx
    flat = hidden_states.reshape(n, H)
    b2 = b.reshape(1, E)
    grid = n // T_BLK

    probs, probs_t, _psum, _cnt, aux = pl.pallas_call(
        _router_body,
        grid=(grid,),
        in_specs=[
            pl.BlockSpec((T_BLK, H), lambda i: (i, 0)),
            pl.BlockSpec((E, H), lambda i: (0, 0)),
            pl.BlockSpec((1, E), lambda i: (0, 0)),
        ],
        out_specs=[
            pl.BlockSpec((T_BLK, E), lambda i: (i, 0)),
            pl.BlockSpec((1, E, T_BLK), lambda i: (i, 0, 0)),
            pl.BlockSpec((1, E), lambda i: (0, 0)),
            pl.BlockSpec((1, E), lambda i: (0, 0)),
            pl.BlockSpec((1, 1), lambda i: (0, 0)),
        ],
        out_shape=[
            jax.ShapeDtypeStruct((n, E), jnp.float32),
            jax.ShapeDtypeStruct((grid, E, T_BLK), jnp.float32),
            jax.ShapeDtypeStruct((1, E), jnp.float32),
            jax.ShapeDtypeStruct((1, E), jnp.float32),
            jax.ShapeDtypeStruct((1, 1), jnp.float32),
        ],
    )(flat, W, b2)

    mesh = plsc.VectorSubcoreMesh(core_axis_name="c", subcore_axis_name="s")
    sc_route = functools.partial(
        pl.kernel,
        mesh=mesh,
        out_type=[
            jax.ShapeDtypeStruct((n,), jnp.float32),
            jax.ShapeDtypeStruct((n,), jnp.float32),
            jax.ShapeDtypeStruct((n,), jnp.int32),
            jax.ShapeDtypeStruct((n,), jnp.int32),
        ],
        scratch_types=[
            pltpu.VMEM((_TPT * E,), jnp.float32),
            pltpu.VMEM((_TPT,), jnp.float32),
            pltpu.VMEM((_TPT,), jnp.float32),
            pltpu.VMEM((_TPT,), jnp.int32),
            pltpu.VMEM((_TPT,), jnp.int32),
        ],
    )(_sc_route_body)
    w1, w2, i1, i2 = sc_route(probs_t.reshape(-1))

    weights = jnp.stack([w1, w2], axis=1)
    indices = jnp.stack([i1, i2], axis=1)
    return (weights, indices, probs, aux[0, 0])


# speedup vs baseline: 1.1356x; 1.1356x over previous
import functools

import jax
import jax.numpy as jnp
from jax import lax
from jax.experimental import pallas as pl
from jax.experimental.pallas import tpu as pltpu
from jax.experimental.pallas import tpu_sc as plsc

E = 64
K = 2
AUX_COEF = 0.01
T_BLK = 1024

_NC = 2
_NS = 16
_NW = _NC * _NS
_TPT = 1024
_G = 4


def _router_body(x_ref, w_ref, b_ref, p_ref, pt_ref, psum_ref, cnt_ref,
                 aux_ref):
    i = pl.program_id(0)
    n_tok_total = pl.num_programs(0) * x_ref.shape[0]
    acc = jax.lax.dot_general(
        x_ref[...], w_ref[...], (((1,), (1,)), ((), ())),
        preferred_element_type=jnp.float32)
    lb = (acc + b_ref[...].astype(jnp.float32)).astype(jnp.bfloat16)
    l = lb.astype(jnp.float32)
    e = jnp.exp(l)
    s = jnp.sum(e, axis=-1, keepdims=True)
    inv = 1.0 / s
    p = e * inv
    p_ref[...] = p
    pt_ref[...] = p.T.reshape(E * T_BLK // 128, 128)

    iota = jax.lax.broadcasted_iota(jnp.int32, p.shape, 1)
    pbits = jax.lax.bitcast_convert_type(p, jnp.int32)
    key = (pbits & ~63) | (63 - iota)
    k1 = jnp.max(key, axis=-1, keepdims=True)
    eq1 = key == k1

    ps = jnp.sum(p, axis=0, keepdims=True)
    cs = jnp.sum(eq1.astype(jnp.float32), axis=0, keepdims=True)

    @pl.when(i == 0)
    def _():
        psum_ref[...] = ps
        cnt_ref[...] = cs

    @pl.when(i > 0)
    def _():
        psum_ref[...] += ps
        cnt_ref[...] += cs

    @pl.when(i == pl.num_programs(0) - 1)
    def _():
        scale = E * AUX_COEF / (float(n_tok_total) * float(n_tok_total))
        aux_ref[...] = jnp.sum(psum_ref[...] * cnt_ref[...],
                               axis=(0, 1), keepdims=True) * scale


def _sc_route_body(pt_hbm, w1_hbm, w2_hbm, i1_hbm, i2_hbm, pt_v,
                   w1_v, w2_v, i1_v, i2_v):
    wid = lax.axis_index("s") * _NC + lax.axis_index("c")
    base = wid * _TPT
    pltpu.sync_copy(pt_hbm.at[pl.ds(wid * (_TPT * E), _TPT * E)], pt_v)
    lanes = lax.iota(jnp.int32, 16)

    def tstep(t, carry_unused):
        tok0 = t * (16 * _G)
        init = []
        for g in range(_G):
            init += [jnp.full((16,), -1.0, jnp.float32),
                     jnp.zeros((16,), jnp.int32),
                     jnp.full((16,), -2.0, jnp.float32),
                     jnp.zeros((16,), jnp.int32)]

        def inner(j, carry):
            cols = jnp.full((16,), j, jnp.int32)
            off = j * _TPT + tok0
            out = []
            for g in range(_G):
                m1, i1, m2, i2 = carry[4 * g:4 * g + 4]
                pj = pt_v[pl.ds(off + 16 * g, 16)]
                gt1 = pj > m1
                gt2 = pj > m2
                m2n = jnp.where(gt1, m1, jnp.where(gt2, pj, m2))
                i2n = jnp.where(gt1, i1, jnp.where(gt2, cols, i2))
                m1n = jnp.where(gt1, pj, m1)
                i1n = jnp.where(gt1, cols, i1)
                out += [m1n, i1n, m2n, i2n]
            return tuple(out)

        res = lax.fori_loop(0, E, inner, tuple(init))
        for g in range(_G):
            m1, i1, m2, i2 = res[4 * g:4 * g + 4]
            e2 = jnp.exp(m2 - m1)
            den = 1.0 + e2
            sl = pl.ds(tok0 + 16 * g, 16)
            w1_v[sl] = 1.0 / den
            w2_v[sl] = e2 / den
            i1_v[sl] = i1
            i2_v[sl] = i2
        return 0

    lax.fori_loop(0, _TPT // (16 * _G), tstep, 0)
    pltpu.sync_copy(w1_v, w1_hbm.at[pl.ds(base, _TPT)])
    pltpu.sync_copy(w2_v, w2_hbm.at[pl.ds(base, _TPT)])
    pltpu.sync_copy(i1_v, i1_hbm.at[pl.ds(base, _TPT)])
    pltpu.sync_copy(i2_v, i2_hbm.at[pl.ds(base, _TPT)])


def kernel(hidden_states, W, b):
    Bx, Sx, H = hidden_states.shape
    n = Bx * Sx
    flat = hidden_states.reshape(n, H)
    b2 = b.reshape(1, E)
    grid = n // T_BLK

    probs, probs_t, _psum, _cnt, aux = pl.pallas_call(
        _router_body,
        grid=(grid,),
        in_specs=[
            pl.BlockSpec((T_BLK, H), lambda i: (i, 0)),
            pl.BlockSpec((E, H), lambda i: (0, 0)),
            pl.BlockSpec((1, E), lambda i: (0, 0)),
        ],
        out_specs=[
            pl.BlockSpec((T_BLK, E), lambda i: (i, 0)),
            pl.BlockSpec((E * T_BLK // 128, 128), lambda i: (i, 0)),
            pl.BlockSpec((1, E), lambda i: (0, 0)),
            pl.BlockSpec((1, E), lambda i: (0, 0)),
            pl.BlockSpec((1, 1), lambda i: (0, 0)),
        ],
        out_shape=[
            jax.ShapeDtypeStruct((n, E), jnp.float32),
            jax.ShapeDtypeStruct((grid * E * T_BLK // 128, 128), jnp.float32),
            jax.ShapeDtypeStruct((1, E), jnp.float32),
            jax.ShapeDtypeStruct((1, E), jnp.float32),
            jax.ShapeDtypeStruct((1, 1), jnp.float32),
        ],
    )(flat, W, b2)

    mesh = plsc.VectorSubcoreMesh(core_axis_name="c", subcore_axis_name="s")
    sc_route = functools.partial(
        pl.kernel,
        mesh=mesh,
        out_type=[
            jax.ShapeDtypeStruct((n,), jnp.float32),
            jax.ShapeDtypeStruct((n,), jnp.float32),
            jax.ShapeDtypeStruct((n,), jnp.int32),
            jax.ShapeDtypeStruct((n,), jnp.int32),
        ],
        scratch_types=[
            pltpu.VMEM((_TPT * E,), jnp.float32),
            pltpu.VMEM((_TPT,), jnp.float32),
            pltpu.VMEM((_TPT,), jnp.float32),
            pltpu.VMEM((_TPT,), jnp.int32),
            pltpu.VMEM((_TPT,), jnp.int32),
        ],
    )(_sc_route_body)
    w1, w2, i1, i2 = sc_route(probs_t.reshape(-1))

    weights = jnp.stack([w1, w2], axis=1)
    indices = jnp.stack([i1, i2], axis=1)
    return (weights, indices, probs, aux[0, 0])
